# Initial kernel scaffold; baseline (speedup 1.0000x reference)
import jax
import jax.numpy as jnp
from jax.experimental import pallas as pl


def kernel(x, edge_index, batch, W1, b1, g1, be1, W2, b2, g2, be2, W3, b3, g3, be3):
    raise NotImplementedError("write your pallas kernel here")


# speedup vs baseline: 14.9547x; 14.9547x over previous
import functools

import jax
import jax.numpy as jnp
from jax import lax
from jax.experimental import pallas as pl
from jax.experimental.pallas import tpu as pltpu
from jax.experimental.pallas import tpu_sc as plsc

F = 128
DEGW = 16
NC, NS = 2, 16
NW = NC * NS
C = 80
RPT = 632
NPAD = NS * RPT
BLK = 1000
EPS = 1e-5


def _sc_mesh():
    return plsc.VectorSubcoreMesh(core_axis_name="c", subcore_axis_name="s")


def _zero_slice(zb_v, acc, base):
    nfull, rem = divmod(RPT, 80)
    for k in range(nfull):
        pltpu.sync_copy(zb_v, acc.at[pl.ds(base + k * 80, 80)])
    if rem:
        pltpu.sync_copy(zb_v.at[pl.ds(0, rem)],
                        acc.at[pl.ds(base + nfull * 80, rem)])


@functools.cache
def _deg_fn(N, E):
    chunks = E // NW // C
    ngrp = chunks // IGRP

    @functools.partial(
        pl.kernel,
        out_type=jax.ShapeDtypeStruct((NC, NPAD, F), jnp.float32),
        mesh=_sc_mesh(),
        scratch_types=[
            pltpu.VMEM((IGRP, C), jnp.int32),
            pltpu.VMEM((C, F), jnp.float32),
            pltpu.VMEM((80, F), jnp.float32),
            pltpu.VMEM_SHARED((NPAD, F), jnp.float32),
        ],
    )
    def deg(dst_hbm, out_hbm, dst_v, ones_v, zb_v, acc):
        cid = lax.axis_index("c")
        sid = lax.axis_index("s")
        wid = sid * NC + cid

        def fill_ones(r, carry):
            for q in range(F // 16):
                ones_v[r, pl.ds(q * 16, 16)] = jnp.ones((16,), jnp.float32)
            return carry

        lax.fori_loop(0, C, fill_ones, 0)

        def fill_zero(r, carry):
            for q in range(F // 16):
                zb_v[r, pl.ds(q * 16, 16)] = jnp.zeros((16,), jnp.float32)
            return carry

        lax.fori_loop(0, 80, fill_zero, 0)

        base = sid * RPT
        _zero_slice(zb_v, acc, base)
        plsc.subcore_barrier()

        def grp(g, carry):
            pltpu.sync_copy(dst_hbm.at[wid, g], dst_v)

            def body(j, carry2):
                pltpu.sync_copy(ones_v, acc.at[dst_v.at[j]], add=True)
                return carry2

            lax.fori_loop(0, IGRP, body, 0)
            return carry

        lax.fori_loop(0, ngrp, grp, 0)
        plsc.subcore_barrier()
        pltpu.sync_copy(acc.at[pl.ds(base, RPT)],
                        out_hbm.at[cid, pl.ds(base, RPT)])

    return deg


IGRP = 25


@functools.cache
def _gather_scatter_fn(N, E):
    chunks = E // NW // C
    ngrp = chunks // IGRP

    @functools.partial(
        pl.kernel,
        out_type=jax.ShapeDtypeStruct((NC, NPAD, F), jnp.float32),
        mesh=_sc_mesh(),
        scratch_types=[
            pltpu.VMEM((IGRP, C), jnp.int32),
            pltpu.VMEM((IGRP, C), jnp.int32),
            pltpu.VMEM((C, F), jnp.float32),
            pltpu.VMEM((80, F), jnp.float32),
            pltpu.VMEM_SHARED((NPAD, F), jnp.float32),
            pltpu.SemaphoreType.DMA,
        ],
    )
    def gs(t_hbm, src_hbm, dst_hbm, out_hbm, src_v, dst_v, rows_v, zb_v, acc,
           sem):
        cid = lax.axis_index("c")
        sid = lax.axis_index("s")
        wid = sid * NC + cid

        def fill_zero(r, carry):
            for q in range(F // 16):
                zb_v[r, pl.ds(q * 16, 16)] = jnp.zeros((16,), jnp.float32)
            return carry

        lax.fori_loop(0, 80, fill_zero, 0)

        base = sid * RPT
        _zero_slice(zb_v, acc, base)
        plsc.subcore_barrier()

        def grp(g, carry):
            pltpu.sync_copy(src_hbm.at[wid, g], src_v)
            pltpu.sync_copy(dst_hbm.at[wid, g], dst_v)

            def body(j, carry2):
                pltpu.async_copy(t_hbm.at[src_v.at[j]], rows_v, sem).wait()
                pltpu.sync_copy(rows_v, acc.at[dst_v.at[j]], add=True)
                return carry2

            lax.fori_loop(0, IGRP, body, 0)
            return carry

        lax.fori_loop(0, ngrp, grp, 0)
        plsc.subcore_barrier()
        pltpu.sync_copy(acc.at[pl.ds(base, RPT)],
                        out_hbm.at[cid, pl.ds(base, RPT)])

    return gs


def _prep_call(x, W1, deg_parts):
    N = x.shape[0]
    nb = N // BLK

    def body(xref, wref, dref, tref, dvref):
        d = dref[...]
        degv = d[0, :, 0:1] + d[1, :, 0:1] + 1.0
        dinv = 1.0 / jnp.sqrt(jnp.maximum(degv, 1.0))
        h = jax.lax.dot_general(xref[...], wref[...], (((1,), (0,)), ((), ())),
                                preferred_element_type=jnp.float32)
        tref[...] = dinv * h
        dvref[...] = jnp.broadcast_to(dinv, (BLK, 8))

    return pl.pallas_call(
        body,
        grid=(nb,),
        in_specs=[
            pl.BlockSpec((BLK, F), lambda i: (i, 0)),
            pl.BlockSpec((F, F), lambda i: (0, 0)),
            pl.BlockSpec((NC, BLK, F), lambda i: (0, i, 0)),
        ],
        out_specs=[
            pl.BlockSpec((BLK, F), lambda i: (i, 0)),
            pl.BlockSpec((BLK, 8), lambda i: (i, 0)),
        ],
        out_shape=[
            jax.ShapeDtypeStruct((N, F), jnp.float32),
            jax.ShapeDtypeStruct((N, 8), jnp.float32),
        ],
    )(x, W1, deg_parts)


def _bn_layer_call(s_parts, t_prev, dinv8, b, g, be, Wn):
    N = t_prev.shape[0]
    nb = N // BLK
    last = Wn is None
    ninv = 1.0 / N

    def body(sref, tref, dvref, bref, gref, beref, *rest):
        if last:
            (uref, stats) = rest
        else:
            (wref, tref_o, stats) = rest
        p = pl.program_id(0)
        i = pl.program_id(1)
        s = sref[...]
        dinv = dvref[...][:, 0:1]
        a = dinv * (s[0] + s[1] + tref[...]) + bref[...]

        @pl.when(p == 0)
        def _():
            @pl.when(i == 0)
            def _():
                stats[...] = jnp.zeros((2, F), jnp.float32)

            stats[0:1, :] = stats[0:1, :] + jnp.sum(a, 0, keepdims=True)
            stats[1:2, :] = stats[1:2, :] + jnp.sum(a * a, 0, keepdims=True)

        @pl.when(p == 1)
        def _():
            mu = stats[0:1, :] * ninv
            var = stats[1:2, :] * ninv - mu * mu
            u = gref[...] * (a - mu) / jnp.sqrt(var + EPS) + beref[...]
            u = jnp.maximum(u, 0.0)
            if last:
                uref[...] = u
            else:
                h = jax.lax.dot_general(u, wref[...], (((1,), (0,)), ((), ())),
                                        preferred_element_type=jnp.float32)
                tref_o[...] = dinv * h

    in_specs = [
        pl.BlockSpec((NC, BLK, F), lambda p, i: (0, i, 0)),
        pl.BlockSpec((BLK, F), lambda p, i: (i, 0)),
        pl.BlockSpec((BLK, 8), lambda p, i: (i, 0)),
        pl.BlockSpec((1, F), lambda p, i: (0, 0)),
        pl.BlockSpec((1, F), lambda p, i: (0, 0)),
        pl.BlockSpec((1, F), lambda p, i: (0, 0)),
    ]
    args = [s_parts, t_prev, dinv8, b.reshape(1, F), g.reshape(1, F),
            be.reshape(1, F)]
    if last:
        out_specs = [pl.BlockSpec((BLK, F), lambda p, i: (i, 0))]
        out_shape = [jax.ShapeDtypeStruct((N, F), jnp.float32)]
    else:
        in_specs.append(pl.BlockSpec((F, F), lambda p, i: (0, 0)))
        args.append(Wn)
        out_specs = [pl.BlockSpec((BLK, F), lambda p, i: (i, 0))]
        out_shape = [jax.ShapeDtypeStruct((N, F), jnp.float32)]

    out = pl.pallas_call(
        body,
        grid=(2, nb),
        in_specs=in_specs,
        out_specs=out_specs,
        out_shape=out_shape,
        scratch_shapes=[pltpu.VMEM((2, F), jnp.float32)],
    )(*args)
    return out[0]


def _pool_call(u, batch_r, G):
    N = u.shape[0]
    nb = N // BLK

    def body(uref, bref, oref, sums, cnts):
        i = pl.program_id(0)

        @pl.when(i == 0)
        def _():
            sums[...] = jnp.zeros((G, F), jnp.float32)
            cnts[...] = jnp.zeros((G, F), jnp.float32)

        seg = jnp.broadcast_to(bref[...][0], (G, BLK))
        ids = lax.broadcasted_iota(jnp.int32, (G, BLK), 0)
        oh = (ids == seg).astype(jnp.float32)
        sums[...] = sums[...] + jax.lax.dot_general(
            oh, uref[...], (((1,), (0,)), ((), ())),
            preferred_element_type=jnp.float32)
        cnts[...] = cnts[...] + jnp.broadcast_to(
            jnp.sum(oh, 1, keepdims=True), (G, F))

        @pl.when(i == nb - 1)
        def _():
            oref[...] = sums[...] / jnp.maximum(cnts[...], 1.0)

    return pl.pallas_call(
        body,
        grid=(nb,),
        in_specs=[
            pl.BlockSpec((BLK, F), lambda i: (i, 0)),
            pl.BlockSpec((1, 1, BLK), lambda i: (i, 0, 0)),
        ],
        out_specs=pl.BlockSpec((G, F), lambda i: (0, 0)),
        out_shape=jax.ShapeDtypeStruct((G, F), jnp.float32),
        scratch_shapes=[pltpu.VMEM((G, F), jnp.float32),
                        pltpu.VMEM((G, F), jnp.float32)],
    )(u, batch_r)


def kernel(x, edge_index, batch, W1, b1, g1, be1, W2, b2, g2, be2,
           W3, b3, g3, be3):
    N = x.shape[0]
    E = edge_index.shape[1]
    G = 64
    src = edge_index[0].reshape(NW, -1, IGRP, C)
    dst = edge_index[1].reshape(NW, -1, IGRP, C)
    batch_r = batch.reshape(N // BLK, 1, BLK)

    deg_parts = _deg_fn(N, E)(dst)
    t, dinv8 = _prep_call(x, W1, deg_parts)

    gs = _gather_scatter_fn(N, E)
    s = gs(t, src, dst)
    t = _bn_layer_call(s, t, dinv8, b1, g1, be1, W2)
    s = gs(t, src, dst)
    t = _bn_layer_call(s, t, dinv8, b2, g2, be2, W3)
    s = gs(t, src, dst)
    u = _bn_layer_call(s, t, dinv8, b3, g3, be3, None)

    return _pool_call(u, batch_r, G)


# speedup vs baseline: 21.5309x; 1.4397x over previous
import functools

import jax
import jax.numpy as jnp
from jax import lax
from jax.experimental import pallas as pl
from jax.experimental.pallas import tpu as pltpu
from jax.experimental.pallas import tpu_sc as plsc

F = 128
DEGW = 16
NC, NS = 2, 16
NW = NC * NS
C = 80
RPT = 632
NPAD = NS * RPT
BLK = 1000
EPS = 1e-5


def _sc_mesh():
    return plsc.VectorSubcoreMesh(core_axis_name="c", subcore_axis_name="s")


def _zero_slice(zb_v, acc, base):
    nfull, rem = divmod(RPT, 80)
    for k in range(nfull):
        pltpu.sync_copy(zb_v, acc.at[pl.ds(base + k * 80, 80)])
    if rem:
        pltpu.sync_copy(zb_v.at[pl.ds(0, rem)],
                        acc.at[pl.ds(base + nfull * 80, rem)])


@functools.cache
def _deg_fn(N, E):
    chunks = E // NW // C
    ngrp = chunks // IGRP

    @functools.partial(
        pl.kernel,
        out_type=jax.ShapeDtypeStruct((NC, NPAD, F), jnp.float32),
        mesh=_sc_mesh(),
        scratch_types=[
            pltpu.VMEM((IGRP, C), jnp.int32),
            pltpu.VMEM((C, F), jnp.float32),
            pltpu.VMEM((80, F), jnp.float32),
            pltpu.VMEM_SHARED((NPAD, F), jnp.float32),
        ],
    )
    def deg(dst_hbm, out_hbm, dst_v, ones_v, zb_v, acc):
        cid = lax.axis_index("c")
        sid = lax.axis_index("s")
        wid = sid * NC + cid

        def fill_ones(r, carry):
            for q in range(F // 16):
                ones_v[r, pl.ds(q * 16, 16)] = jnp.ones((16,), jnp.float32)
            return carry

        lax.fori_loop(0, C, fill_ones, 0)

        def fill_zero(r, carry):
            for q in range(F // 16):
                zb_v[r, pl.ds(q * 16, 16)] = jnp.zeros((16,), jnp.float32)
            return carry

        lax.fori_loop(0, 80, fill_zero, 0)

        base = sid * RPT
        _zero_slice(zb_v, acc, base)
        plsc.subcore_barrier()

        def grp(g, carry):
            pltpu.sync_copy(dst_hbm.at[wid, g], dst_v)

            def body(j, carry2):
                pltpu.sync_copy(ones_v, acc.at[dst_v.at[j]], add=True)
                return carry2

            lax.fori_loop(0, IGRP, body, 0)
            return carry

        lax.fori_loop(0, ngrp, grp, 0)
        plsc.subcore_barrier()
        pltpu.sync_copy(acc.at[pl.ds(base, RPT)],
                        out_hbm.at[cid, pl.ds(base, RPT)])

    return deg


IGRP = 25


@functools.cache
def _gather_scatter_fn(N, E):
    chunks = E // NW // C
    ngrp = chunks // IGRP

    @functools.partial(
        pl.kernel,
        out_type=jax.ShapeDtypeStruct((NC, NPAD, F), jnp.float32),
        mesh=_sc_mesh(),
        scratch_types=[
            pltpu.VMEM((IGRP, C), jnp.int32),
            pltpu.VMEM((IGRP, C), jnp.int32),
            pltpu.VMEM((C, F), jnp.float32),
            pltpu.VMEM((C, F), jnp.float32),
            pltpu.VMEM((80, F), jnp.float32),
            pltpu.VMEM_SHARED((NPAD, F), jnp.float32),
            pltpu.SemaphoreType.DMA,
            pltpu.SemaphoreType.DMA,
        ],
    )
    def gs(t_hbm, src_hbm, dst_hbm, out_hbm, src_v, dst_v, rows0, rows1,
           zb_v, acc, sem0, sem1):
        cid = lax.axis_index("c")
        sid = lax.axis_index("s")
        wid = sid * NC + cid

        def fill_zero(r, carry):
            for q in range(F // 16):
                zb_v[r, pl.ds(q * 16, 16)] = jnp.zeros((16,), jnp.float32)
            return carry

        lax.fori_loop(0, 80, fill_zero, 0)

        base = sid * RPT
        _zero_slice(zb_v, acc, base)
        plsc.subcore_barrier()

        def grp(g, carry):
            pltpu.sync_copy(src_hbm.at[wid, g], src_v)
            pltpu.sync_copy(dst_hbm.at[wid, g], dst_v)
            pltpu.async_copy(t_hbm.at[src_v.at[0]], rows0, sem0)

            def body(j, carry2):
                nxt = j + 1

                @pl.when(jnp.logical_and(nxt < IGRP, nxt % 2 == 1))
                def _():
                    pltpu.async_copy(t_hbm.at[src_v.at[nxt]], rows1, sem1)

                @pl.when(jnp.logical_and(nxt < IGRP, nxt % 2 == 0))
                def _():
                    pltpu.async_copy(t_hbm.at[src_v.at[nxt]], rows0, sem0)

                @pl.when(j % 2 == 0)
                def _():
                    pltpu.make_async_copy(t_hbm.at[src_v.at[j]], rows0,
                                          sem0).wait()
                    pltpu.sync_copy(rows0, acc.at[dst_v.at[j]], add=True)

                @pl.when(j % 2 == 1)
                def _():
                    pltpu.make_async_copy(t_hbm.at[src_v.at[j]], rows1,
                                          sem1).wait()
                    pltpu.sync_copy(rows1, acc.at[dst_v.at[j]], add=True)

                return carry2

            lax.fori_loop(0, IGRP, body, 0)
            return carry

        lax.fori_loop(0, ngrp, grp, 0)
        plsc.subcore_barrier()
        pltpu.sync_copy(acc.at[pl.ds(base, RPT)],
                        out_hbm.at[cid, pl.ds(base, RPT)])

    return gs


def _prep_call(x, W1, deg_parts):
    N = x.shape[0]
    nb = N // BLK

    def body(xref, wref, dref, tref, dvref):
        d = dref[...]
        degv = d[0, :, 0:1] + d[1, :, 0:1] + 1.0
        dinv = 1.0 / jnp.sqrt(jnp.maximum(degv, 1.0))
        h = jax.lax.dot_general(xref[...], wref[...], (((1,), (0,)), ((), ())),
                                preferred_element_type=jnp.float32)
        tref[...] = dinv * h
        dvref[...] = jnp.broadcast_to(dinv, (BLK, 8))

    return pl.pallas_call(
        body,
        grid=(nb,),
        in_specs=[
            pl.BlockSpec((BLK, F), lambda i: (i, 0)),
            pl.BlockSpec((F, F), lambda i: (0, 0)),
            pl.BlockSpec((NC, BLK, F), lambda i: (0, i, 0)),
        ],
        out_specs=[
            pl.BlockSpec((BLK, F), lambda i: (i, 0)),
            pl.BlockSpec((BLK, 8), lambda i: (i, 0)),
        ],
        out_shape=[
            jax.ShapeDtypeStruct((N, F), jnp.float32),
            jax.ShapeDtypeStruct((N, 8), jnp.float32),
        ],
    )(x, W1, deg_parts)


def _bn_layer_call(s_parts, t_prev, dinv8, b, g, be, Wn):
    N = t_prev.shape[0]
    nb = N // BLK
    last = Wn is None
    ninv = 1.0 / N

    def body(sref, tref, dvref, bref, gref, beref, *rest):
        if last:
            (uref, stats) = rest
        else:
            (wref, tref_o, stats) = rest
        p = pl.program_id(0)
        i = pl.program_id(1)
        s = sref[...]
        dinv = dvref[...][:, 0:1]
        a = dinv * (s[0] + s[1] + tref[...]) + bref[...]

        @pl.when(p == 0)
        def _():
            @pl.when(i == 0)
            def _():
                stats[...] = jnp.zeros((2, F), jnp.float32)

            stats[0:1, :] = stats[0:1, :] + jnp.sum(a, 0, keepdims=True)
            stats[1:2, :] = stats[1:2, :] + jnp.sum(a * a, 0, keepdims=True)

        @pl.when(p == 1)
        def _():
            mu = stats[0:1, :] * ninv
            var = stats[1:2, :] * ninv - mu * mu
            u = gref[...] * (a - mu) / jnp.sqrt(var + EPS) + beref[...]
            u = jnp.maximum(u, 0.0)
            if last:
                uref[...] = u
            else:
                h = jax.lax.dot_general(u, wref[...], (((1,), (0,)), ((), ())),
                                        preferred_element_type=jnp.float32)
                tref_o[...] = dinv * h

    in_specs = [
        pl.BlockSpec((NC, BLK, F), lambda p, i: (0, i, 0)),
        pl.BlockSpec((BLK, F), lambda p, i: (i, 0)),
        pl.BlockSpec((BLK, 8), lambda p, i: (i, 0)),
        pl.BlockSpec((1, F), lambda p, i: (0, 0)),
        pl.BlockSpec((1, F), lambda p, i: (0, 0)),
        pl.BlockSpec((1, F), lambda p, i: (0, 0)),
    ]
    args = [s_parts, t_prev, dinv8, b.reshape(1, F), g.reshape(1, F),
            be.reshape(1, F)]
    if last:
        out_specs = [pl.BlockSpec((BLK, F), lambda p, i: (i, 0))]
        out_shape = [jax.ShapeDtypeStruct((N, F), jnp.float32)]
    else:
        in_specs.append(pl.BlockSpec((F, F), lambda p, i: (0, 0)))
        args.append(Wn)
        out_specs = [pl.BlockSpec((BLK, F), lambda p, i: (i, 0))]
        out_shape = [jax.ShapeDtypeStruct((N, F), jnp.float32)]

    out = pl.pallas_call(
        body,
        grid=(2, nb),
        in_specs=in_specs,
        out_specs=out_specs,
        out_shape=out_shape,
        scratch_shapes=[pltpu.VMEM((2, F), jnp.float32)],
    )(*args)
    return out[0]


def _pool_call(u, batch_r, G):
    N = u.shape[0]
    nb = N // BLK

    def body(uref, bref, oref, sums, cnts):
        i = pl.program_id(0)

        @pl.when(i == 0)
        def _():
            sums[...] = jnp.zeros((G, F), jnp.float32)
            cnts[...] = jnp.zeros((G, F), jnp.float32)

        seg = jnp.broadcast_to(bref[...][0], (G, BLK))
        ids = lax.broadcasted_iota(jnp.int32, (G, BLK), 0)
        oh = (ids == seg).astype(jnp.float32)
        sums[...] = sums[...] + jax.lax.dot_general(
            oh, uref[...], (((1,), (0,)), ((), ())),
            preferred_element_type=jnp.float32)
        cnts[...] = cnts[...] + jnp.broadcast_to(
            jnp.sum(oh, 1, keepdims=True), (G, F))

        @pl.when(i == nb - 1)
        def _():
            oref[...] = sums[...] / jnp.maximum(cnts[...], 1.0)

    return pl.pallas_call(
        body,
        grid=(nb,),
        in_specs=[
            pl.BlockSpec((BLK, F), lambda i: (i, 0)),
            pl.BlockSpec((1, 1, BLK), lambda i: (i, 0, 0)),
        ],
        out_specs=pl.BlockSpec((G, F), lambda i: (0, 0)),
        out_shape=jax.ShapeDtypeStruct((G, F), jnp.float32),
        scratch_shapes=[pltpu.VMEM((G, F), jnp.float32),
                        pltpu.VMEM((G, F), jnp.float32)],
    )(u, batch_r)


def kernel(x, edge_index, batch, W1, b1, g1, be1, W2, b2, g2, be2,
           W3, b3, g3, be3):
    N = x.shape[0]
    E = edge_index.shape[1]
    G = 64
    src = edge_index[0].reshape(NW, -1, IGRP, C)
    dst = edge_index[1].reshape(NW, -1, IGRP, C)
    batch_r = batch.reshape(N // BLK, 1, BLK)

    deg_parts = _deg_fn(N, E)(dst)
    t, dinv8 = _prep_call(x, W1, deg_parts)

    gs = _gather_scatter_fn(N, E)
    s = gs(t, src, dst)
    t = _bn_layer_call(s, t, dinv8, b1, g1, be1, W2)
    s = gs(t, src, dst)
    t = _bn_layer_call(s, t, dinv8, b2, g2, be2, W3)
    s = gs(t, src, dst)
    u = _bn_layer_call(s, t, dinv8, b3, g3, be3, None)

    return _pool_call(u, batch_r, G)


# speedup vs baseline: 23.8857x; 1.1094x over previous
import functools

import jax
import jax.numpy as jnp
from jax import lax
from jax.experimental import pallas as pl
from jax.experimental.pallas import tpu as pltpu
from jax.experimental.pallas import tpu_sc as plsc

F = 128
DEGW = 16
NC, NS = 2, 16
NW = NC * NS
C = 80
RPT = 632
NPAD = NS * RPT
BLK = 1000
EPS = 1e-5


def _sc_mesh():
    return plsc.VectorSubcoreMesh(core_axis_name="c", subcore_axis_name="s")


def _zero_slice(zb_v, acc, base):
    nfull, rem = divmod(RPT, 80)
    for k in range(nfull):
        pltpu.sync_copy(zb_v, acc.at[pl.ds(base + k * 80, 80)])
    if rem:
        pltpu.sync_copy(zb_v.at[pl.ds(0, rem)],
                        acc.at[pl.ds(base + nfull * 80, rem)])


@functools.cache
def _deg_fn(N, E):
    chunks = E // NW // C
    ngrp = chunks // IGRP

    @functools.partial(
        pl.kernel,
        out_type=jax.ShapeDtypeStruct((NC, NPAD, F), jnp.float32),
        mesh=_sc_mesh(),
        scratch_types=[
            pltpu.VMEM((IGRP, C), jnp.int32),
            pltpu.VMEM((C, F), jnp.float32),
            pltpu.VMEM((80, F), jnp.float32),
            pltpu.VMEM_SHARED((NPAD, F), jnp.float32),
            pltpu.SemaphoreType.DMA,
        ],
    )
    def deg(dst_hbm, out_hbm, dst_v, ones_v, zb_v, acc, sem):
        cid = lax.axis_index("c")
        sid = lax.axis_index("s")
        wid = sid * NC + cid

        def fill_ones(r, carry):
            for q in range(F // 16):
                ones_v[r, pl.ds(q * 16, 16)] = jnp.ones((16,), jnp.float32)
            return carry

        lax.fori_loop(0, C, fill_ones, 0)

        def fill_zero(r, carry):
            for q in range(F // 16):
                zb_v[r, pl.ds(q * 16, 16)] = jnp.zeros((16,), jnp.float32)
            return carry

        lax.fori_loop(0, 80, fill_zero, 0)

        base = sid * RPT
        _zero_slice(zb_v, acc, base)
        plsc.subcore_barrier()

        def grp(g, carry):
            pltpu.sync_copy(dst_hbm.at[wid, g], dst_v)

            def fire(j, carry2):
                pltpu.async_copy(ones_v, acc.at[dst_v.at[j]], sem, add=True)
                return carry2

            lax.fori_loop(0, IGRP, fire, 0)

            def drain(j, carry2):
                pltpu.make_async_copy(ones_v, acc.at[dst_v.at[j]], sem).wait()
                return carry2

            lax.fori_loop(0, IGRP, drain, 0)
            return carry

        lax.fori_loop(0, ngrp, grp, 0)
        plsc.subcore_barrier()
        pltpu.sync_copy(acc.at[pl.ds(base, RPT)],
                        out_hbm.at[cid, pl.ds(base, RPT)])

    return deg


IGRP = 25


@functools.cache
def _gather_scatter_fn(N, E):
    chunks = E // NW // C
    ngrp = chunks // IGRP

    @functools.partial(
        pl.kernel,
        out_type=jax.ShapeDtypeStruct((NC, NPAD, F), jnp.float32),
        mesh=_sc_mesh(),
        scratch_types=[
            pltpu.VMEM((IGRP, C), jnp.int32),
            pltpu.VMEM((IGRP, C), jnp.int32),
            pltpu.VMEM((C, F), jnp.float32),
            pltpu.VMEM((C, F), jnp.float32),
            pltpu.VMEM((C, F), jnp.float32),
            pltpu.VMEM((80, F), jnp.float32),
            pltpu.VMEM_SHARED((NPAD, F), jnp.float32),
            pltpu.SemaphoreType.DMA,
            pltpu.SemaphoreType.DMA,
            pltpu.SemaphoreType.DMA,
            pltpu.SemaphoreType.DMA,
            pltpu.SemaphoreType.DMA,
            pltpu.SemaphoreType.DMA,
        ],
    )
    def gs(t_hbm, src_hbm, dst_hbm, out_hbm, src_v, dst_v, rows0, rows1,
           rows2, zb_v, acc, gsem0, gsem1, gsem2, ssem0, ssem1, ssem2):
        cid = lax.axis_index("c")
        sid = lax.axis_index("s")
        wid = sid * NC + cid
        rows = (rows0, rows1, rows2)
        gsem = (gsem0, gsem1, gsem2)
        ssem = (ssem0, ssem1, ssem2)

        def fill_zero(r, carry):
            for q in range(F // 16):
                zb_v[r, pl.ds(q * 16, 16)] = jnp.zeros((16,), jnp.float32)
            return carry

        lax.fori_loop(0, 80, fill_zero, 0)

        base = sid * RPT
        _zero_slice(zb_v, acc, base)
        plsc.subcore_barrier()

        def grp(g, carry):
            pltpu.sync_copy(src_hbm.at[wid, g], src_v)
            pltpu.sync_copy(dst_hbm.at[wid, g], dst_v)
            pltpu.async_copy(t_hbm.at[src_v.at[0]], rows0, gsem0)
            pltpu.async_copy(t_hbm.at[src_v.at[1]], rows1, gsem1)

            def body(j, carry2):
                for b in range(3):
                    @pl.when(j % 3 == b)
                    def _(b=b):
                        pltpu.make_async_copy(t_hbm.at[src_v.at[j]], rows[b],
                                              gsem[b]).wait()
                        pltpu.async_copy(rows[b], acc.at[dst_v.at[j]],
                                         ssem[b], add=True)
                        b2 = (b + 2) % 3

                        @pl.when(j + 2 < IGRP)
                        def _():
                            @pl.when(j >= 1)
                            def _():
                                pltpu.make_async_copy(
                                    rows[b2], acc.at[dst_v.at[j - 1]],
                                    ssem[b2]).wait()

                            pltpu.async_copy(t_hbm.at[src_v.at[j + 2]],
                                             rows[b2], gsem[b2])

                return carry2

            lax.fori_loop(0, IGRP, body, 0)
            for k in range(IGRP - 3, IGRP):
                pltpu.make_async_copy(rows[k % 3], acc.at[dst_v.at[k]],
                                      ssem[k % 3]).wait()
            return carry

        lax.fori_loop(0, ngrp, grp, 0)
        plsc.subcore_barrier()
        pltpu.sync_copy(acc.at[pl.ds(base, RPT)],
                        out_hbm.at[cid, pl.ds(base, RPT)])

    return gs


def _prep_call(x, W1, deg_parts):
    N = x.shape[0]
    nb = N // BLK

    def body(xref, wref, dref, tref, dvref):
        d = dref[...]
        degv = d[0, :, 0:1] + d[1, :, 0:1] + 1.0
        dinv = 1.0 / jnp.sqrt(jnp.maximum(degv, 1.0))
        h = jax.lax.dot_general(xref[...], wref[...], (((1,), (0,)), ((), ())),
                                preferred_element_type=jnp.float32)
        tref[...] = dinv * h
        dvref[...] = jnp.broadcast_to(dinv, (BLK, 8))

    return pl.pallas_call(
        body,
        grid=(nb,),
        in_specs=[
            pl.BlockSpec((BLK, F), lambda i: (i, 0)),
            pl.BlockSpec((F, F), lambda i: (0, 0)),
            pl.BlockSpec((NC, BLK, F), lambda i: (0, i, 0)),
        ],
        out_specs=[
            pl.BlockSpec((BLK, F), lambda i: (i, 0)),
            pl.BlockSpec((BLK, 8), lambda i: (i, 0)),
        ],
        out_shape=[
            jax.ShapeDtypeStruct((N, F), jnp.float32),
            jax.ShapeDtypeStruct((N, 8), jnp.float32),
        ],
    )(x, W1, deg_parts)


def _bn_layer_call(s_parts, t_prev, dinv8, b, g, be, Wn):
    N = t_prev.shape[0]
    nb = N // BLK
    last = Wn is None
    ninv = 1.0 / N

    def body(sref, tref, dvref, bref, gref, beref, *rest):
        if last:
            (uref, stats) = rest
        else:
            (wref, tref_o, stats) = rest
        p = pl.program_id(0)
        i = pl.program_id(1)
        s = sref[...]
        dinv = dvref[...][:, 0:1]
        a = dinv * (s[0] + s[1] + tref[...]) + bref[...]

        @pl.when(p == 0)
        def _():
            @pl.when(i == 0)
            def _():
                stats[...] = jnp.zeros((2, F), jnp.float32)

            stats[0:1, :] = stats[0:1, :] + jnp.sum(a, 0, keepdims=True)
            stats[1:2, :] = stats[1:2, :] + jnp.sum(a * a, 0, keepdims=True)

        @pl.when(p == 1)
        def _():
            mu = stats[0:1, :] * ninv
            var = stats[1:2, :] * ninv - mu * mu
            u = gref[...] * (a - mu) / jnp.sqrt(var + EPS) + beref[...]
            u = jnp.maximum(u, 0.0)
            if last:
                uref[...] = u
            else:
                h = jax.lax.dot_general(u, wref[...], (((1,), (0,)), ((), ())),
                                        preferred_element_type=jnp.float32)
                tref_o[...] = dinv * h

    in_specs = [
        pl.BlockSpec((NC, BLK, F), lambda p, i: (0, i, 0)),
        pl.BlockSpec((BLK, F), lambda p, i: (i, 0)),
        pl.BlockSpec((BLK, 8), lambda p, i: (i, 0)),
        pl.BlockSpec((1, F), lambda p, i: (0, 0)),
        pl.BlockSpec((1, F), lambda p, i: (0, 0)),
        pl.BlockSpec((1, F), lambda p, i: (0, 0)),
    ]
    args = [s_parts, t_prev, dinv8, b.reshape(1, F), g.reshape(1, F),
            be.reshape(1, F)]
    if last:
        out_specs = [pl.BlockSpec((BLK, F), lambda p, i: (i, 0))]
        out_shape = [jax.ShapeDtypeStruct((N, F), jnp.float32)]
    else:
        in_specs.append(pl.BlockSpec((F, F), lambda p, i: (0, 0)))
        args.append(Wn)
        out_specs = [pl.BlockSpec((BLK, F), lambda p, i: (i, 0))]
        out_shape = [jax.ShapeDtypeStruct((N, F), jnp.float32)]

    out = pl.pallas_call(
        body,
        grid=(2, nb),
        in_specs=in_specs,
        out_specs=out_specs,
        out_shape=out_shape,
        scratch_shapes=[pltpu.VMEM((2, F), jnp.float32)],
    )(*args)
    return out[0]


def _pool_call(u, batch_r, G):
    N = u.shape[0]
    nb = N // BLK

    def body(uref, bref, oref, sums, cnts):
        i = pl.program_id(0)

        @pl.when(i == 0)
        def _():
            sums[...] = jnp.zeros((G, F), jnp.float32)
            cnts[...] = jnp.zeros((G, F), jnp.float32)

        seg = jnp.broadcast_to(bref[...][0], (G, BLK))
        ids = lax.broadcasted_iota(jnp.int32, (G, BLK), 0)
        oh = (ids == seg).astype(jnp.float32)
        sums[...] = sums[...] + jax.lax.dot_general(
            oh, uref[...], (((1,), (0,)), ((), ())),
            preferred_element_type=jnp.float32)
        cnts[...] = cnts[...] + jnp.broadcast_to(
            jnp.sum(oh, 1, keepdims=True), (G, F))

        @pl.when(i == nb - 1)
        def _():
            oref[...] = sums[...] / jnp.maximum(cnts[...], 1.0)

    return pl.pallas_call(
        body,
        grid=(nb,),
        in_specs=[
            pl.BlockSpec((BLK, F), lambda i: (i, 0)),
            pl.BlockSpec((1, 1, BLK), lambda i: (i, 0, 0)),
        ],
        out_specs=pl.BlockSpec((G, F), lambda i: (0, 0)),
        out_shape=jax.ShapeDtypeStruct((G, F), jnp.float32),
        scratch_shapes=[pltpu.VMEM((G, F), jnp.float32),
                        pltpu.VMEM((G, F), jnp.float32)],
    )(u, batch_r)


def kernel(x, edge_index, batch, W1, b1, g1, be1, W2, b2, g2, be2,
           W3, b3, g3, be3):
    N = x.shape[0]
    E = edge_index.shape[1]
    G = 64
    src = edge_index[0].reshape(NW, -1, IGRP, C)
    dst = edge_index[1].reshape(NW, -1, IGRP, C)
    batch_r = batch.reshape(N // BLK, 1, BLK)

    deg_parts = _deg_fn(N, E)(dst)
    t, dinv8 = _prep_call(x, W1, deg_parts)

    gs = _gather_scatter_fn(N, E)
    s = gs(t, src, dst)
    t = _bn_layer_call(s, t, dinv8, b1, g1, be1, W2)
    s = gs(t, src, dst)
    t = _bn_layer_call(s, t, dinv8, b2, g2, be2, W3)
    s = gs(t, src, dst)
    u = _bn_layer_call(s, t, dinv8, b3, g3, be3, None)

    return _pool_call(u, batch_r, G)


# speedup vs baseline: 24.3107x; 1.0178x over previous
import functools

import jax
import jax.numpy as jnp
from jax import lax
from jax.experimental import pallas as pl
from jax.experimental.pallas import tpu as pltpu
from jax.experimental.pallas import tpu_sc as plsc

F = 128
DEGW = 16
NC, NS = 2, 16
NW = NC * NS
C = 80
RPT = 632
NPAD = NS * RPT
BLK = 1000
EPS = 1e-5


def _sc_mesh():
    return plsc.VectorSubcoreMesh(core_axis_name="c", subcore_axis_name="s")


def _zero_slice(zb_v, acc, base):
    nfull, rem = divmod(RPT, 80)
    for k in range(nfull):
        pltpu.sync_copy(zb_v, acc.at[pl.ds(base + k * 80, 80)])
    if rem:
        pltpu.sync_copy(zb_v.at[pl.ds(0, rem)],
                        acc.at[pl.ds(base + nfull * 80, rem)])


@functools.cache
def _deg_fn(N, E):
    chunks = E // NW // C
    ngrp = chunks // IGRP

    @functools.partial(
        pl.kernel,
        out_type=jax.ShapeDtypeStruct((NC, NPAD, F), jnp.float32),
        mesh=_sc_mesh(),
        scratch_types=[
            pltpu.VMEM((IGRP, C), jnp.int32),
            pltpu.VMEM((C, F), jnp.float32),
            pltpu.VMEM((80, F), jnp.float32),
            pltpu.VMEM_SHARED((NPAD, F), jnp.float32),
            pltpu.SemaphoreType.DMA,
        ],
    )
    def deg(dst_hbm, out_hbm, dst_v, ones_v, zb_v, acc, sem):
        cid = lax.axis_index("c")
        sid = lax.axis_index("s")
        wid = sid * NC + cid

        def fill_ones(r, carry):
            for q in range(F // 16):
                ones_v[r, pl.ds(q * 16, 16)] = jnp.ones((16,), jnp.float32)
            return carry

        lax.fori_loop(0, C, fill_ones, 0)

        def fill_zero(r, carry):
            for q in range(F // 16):
                zb_v[r, pl.ds(q * 16, 16)] = jnp.zeros((16,), jnp.float32)
            return carry

        lax.fori_loop(0, 80, fill_zero, 0)

        base = sid * RPT
        _zero_slice(zb_v, acc, base)
        plsc.subcore_barrier()

        def grp(g, carry):
            pltpu.sync_copy(dst_hbm.at[wid, g], dst_v)

            def fire(j, carry2):
                pltpu.async_copy(ones_v, acc.at[dst_v.at[j]], sem, add=True)
                return carry2

            lax.fori_loop(0, IGRP, fire, 0)

            def drain(j, carry2):
                pltpu.make_async_copy(ones_v, acc.at[dst_v.at[j]], sem).wait()
                return carry2

            lax.fori_loop(0, IGRP, drain, 0)
            return carry

        lax.fori_loop(0, ngrp, grp, 0)
        plsc.subcore_barrier()
        pltpu.sync_copy(acc.at[pl.ds(base, RPT)],
                        out_hbm.at[cid, pl.ds(base, RPT)])

    return deg


IGRP = 25


@functools.cache
def _gather_scatter_fn(N, E):
    chunks = E // NW // C
    ngrp = chunks // IGRP

    @functools.partial(
        pl.kernel,
        out_type=jax.ShapeDtypeStruct((NC, NPAD, F), jnp.float32),
        mesh=_sc_mesh(),
        scratch_types=[
            pltpu.VMEM((IGRP, C), jnp.int32),
            pltpu.VMEM((IGRP, C), jnp.int32),
            pltpu.VMEM((C, F), jnp.float32),
            pltpu.VMEM((C, F), jnp.float32),
            pltpu.VMEM((C, F), jnp.float32),
            pltpu.VMEM((80, F), jnp.float32),
            pltpu.VMEM_SHARED((NPAD, F), jnp.float32),
            pltpu.SemaphoreType.DMA,
            pltpu.SemaphoreType.DMA,
            pltpu.SemaphoreType.DMA,
            pltpu.SemaphoreType.DMA,
            pltpu.SemaphoreType.DMA,
            pltpu.SemaphoreType.DMA,
        ],
    )
    def gs(t_hbm, src_hbm, dst_hbm, out_hbm, src_v, dst_v, rows0, rows1,
           rows2, zb_v, acc, gsem0, gsem1, gsem2, ssem0, ssem1, ssem2):
        cid = lax.axis_index("c")
        sid = lax.axis_index("s")
        wid = sid * NC + cid
        rows = (rows0, rows1, rows2)
        gsem = (gsem0, gsem1, gsem2)
        ssem = (ssem0, ssem1, ssem2)

        def fill_zero(r, carry):
            for q in range(F // 16):
                zb_v[r, pl.ds(q * 16, 16)] = jnp.zeros((16,), jnp.float32)
            return carry

        lax.fori_loop(0, 80, fill_zero, 0)

        base = sid * RPT
        _zero_slice(zb_v, acc, base)
        plsc.subcore_barrier()

        def grp(g, carry):
            pltpu.sync_copy(src_hbm.at[wid, g], src_v)
            pltpu.sync_copy(dst_hbm.at[wid, g], dst_v)
            pltpu.async_copy(t_hbm.at[src_v.at[0]], rows0, gsem0)
            pltpu.async_copy(t_hbm.at[src_v.at[1]], rows1, gsem1)

            def body(j, carry2):
                for b in range(3):
                    @pl.when(j % 3 == b)
                    def _(b=b):
                        pltpu.make_async_copy(t_hbm.at[src_v.at[j]], rows[b],
                                              gsem[b]).wait()
                        pltpu.async_copy(rows[b], acc.at[dst_v.at[j]],
                                         ssem[b], add=True)
                        b2 = (b + 2) % 3

                        @pl.when(j + 2 < IGRP)
                        def _():
                            @pl.when(j >= 1)
                            def _():
                                pltpu.make_async_copy(
                                    rows[b2], acc.at[dst_v.at[j - 1]],
                                    ssem[b2]).wait()

                            pltpu.async_copy(t_hbm.at[src_v.at[j + 2]],
                                             rows[b2], gsem[b2])

                return carry2

            lax.fori_loop(0, IGRP, body, 0)
            for k in range(IGRP - 3, IGRP):
                pltpu.make_async_copy(rows[k % 3], acc.at[dst_v.at[k]],
                                      ssem[k % 3]).wait()
            return carry

        lax.fori_loop(0, ngrp, grp, 0)
        plsc.subcore_barrier()
        pltpu.sync_copy(acc.at[pl.ds(base, RPT)],
                        out_hbm.at[cid, pl.ds(base, RPT)])

    return gs


def _prep_call(x, W1, deg_parts):
    N = x.shape[0]
    nb = N // BLK

    def body(xref, wref, dref, tref, dvref):
        d = dref[...]
        degv = d[0, :, 0:1] + d[1, :, 0:1] + 1.0
        dinv = 1.0 / jnp.sqrt(jnp.maximum(degv, 1.0))
        h = jax.lax.dot_general(xref[...], wref[...], (((1,), (0,)), ((), ())),
                                preferred_element_type=jnp.float32)
        tref[...] = dinv * h
        dvref[...] = jnp.broadcast_to(dinv, (BLK, 8))

    return pl.pallas_call(
        body,
        grid=(nb,),
        in_specs=[
            pl.BlockSpec((BLK, F), lambda i: (i, 0)),
            pl.BlockSpec((F, F), lambda i: (0, 0)),
            pl.BlockSpec((NC, BLK, F), lambda i: (0, i, 0)),
        ],
        out_specs=[
            pl.BlockSpec((BLK, F), lambda i: (i, 0)),
            pl.BlockSpec((BLK, 8), lambda i: (i, 0)),
        ],
        out_shape=[
            jax.ShapeDtypeStruct((N, F), jnp.float32),
            jax.ShapeDtypeStruct((N, 8), jnp.float32),
        ],
    )(x, W1, deg_parts)


def _bn_layer_call(s_parts, t_prev, dinv8, b, g, be, Wn, batch_r=None, G=64):
    N = t_prev.shape[0]
    nb = N // BLK
    last = Wn is None
    ninv = 1.0 / N

    def body(sref, tref, dvref, bref, gref, beref, *rest):
        if last:
            (batchref, oref, stats, sums, cnts) = rest
        else:
            (wref, tref_o, stats) = rest
        p = pl.program_id(0)
        i = pl.program_id(1)
        s = sref[...]
        dinv = dvref[...][:, 0:1]
        a = dinv * (s[0] + s[1] + tref[...]) + bref[...]

        @pl.when(p == 0)
        def _():
            @pl.when(i == 0)
            def _():
                stats[...] = jnp.zeros((2, F), jnp.float32)

            stats[0:1, :] = stats[0:1, :] + jnp.sum(a, 0, keepdims=True)
            stats[1:2, :] = stats[1:2, :] + jnp.sum(a * a, 0, keepdims=True)

        @pl.when(p == 1)
        def _():
            mu = stats[0:1, :] * ninv
            var = stats[1:2, :] * ninv - mu * mu
            u = gref[...] * (a - mu) / jnp.sqrt(var + EPS) + beref[...]
            u = jnp.maximum(u, 0.0)
            if last:
                @pl.when(i == 0)
                def _():
                    sums[...] = jnp.zeros((G, F), jnp.float32)
                    cnts[...] = jnp.zeros((G, F), jnp.float32)

                seg = jnp.broadcast_to(batchref[...][0], (G, BLK))
                ids = lax.broadcasted_iota(jnp.int32, (G, BLK), 0)
                oh = (ids == seg).astype(jnp.float32)
                sums[...] = sums[...] + jax.lax.dot_general(
                    oh, u, (((1,), (0,)), ((), ())),
                    preferred_element_type=jnp.float32)
                cnts[...] = cnts[...] + jnp.broadcast_to(
                    jnp.sum(oh, 1, keepdims=True), (G, F))

                @pl.when(i == nb - 1)
                def _():
                    oref[...] = sums[...] / jnp.maximum(cnts[...], 1.0)
            else:
                h = jax.lax.dot_general(u, wref[...], (((1,), (0,)), ((), ())),
                                        preferred_element_type=jnp.float32)
                tref_o[...] = dinv * h

    in_specs = [
        pl.BlockSpec((NC, BLK, F), lambda p, i: (0, i, 0)),
        pl.BlockSpec((BLK, F), lambda p, i: (i, 0)),
        pl.BlockSpec((BLK, 8), lambda p, i: (i, 0)),
        pl.BlockSpec((1, F), lambda p, i: (0, 0)),
        pl.BlockSpec((1, F), lambda p, i: (0, 0)),
        pl.BlockSpec((1, F), lambda p, i: (0, 0)),
    ]
    args = [s_parts, t_prev, dinv8, b.reshape(1, F), g.reshape(1, F),
            be.reshape(1, F)]
    if last:
        in_specs.append(pl.BlockSpec((1, 1, BLK), lambda p, i: (i, 0, 0)))
        args.append(batch_r)
        out_specs = [pl.BlockSpec((G, F), lambda p, i: (0, 0))]
        out_shape = [jax.ShapeDtypeStruct((G, F), jnp.float32)]
        scratch = [pltpu.VMEM((2, F), jnp.float32),
                   pltpu.VMEM((G, F), jnp.float32),
                   pltpu.VMEM((G, F), jnp.float32)]
    else:
        in_specs.append(pl.BlockSpec((F, F), lambda p, i: (0, 0)))
        args.append(Wn)
        out_specs = [pl.BlockSpec((BLK, F), lambda p, i: (i, 0))]
        out_shape = [jax.ShapeDtypeStruct((N, F), jnp.float32)]
        scratch = [pltpu.VMEM((2, F), jnp.float32)]

    out = pl.pallas_call(
        body,
        grid=(2, nb),
        in_specs=in_specs,
        out_specs=out_specs,
        out_shape=out_shape,
        scratch_shapes=scratch,
    )(*args)
    return out[0]


def kernel(x, edge_index, batch, W1, b1, g1, be1, W2, b2, g2, be2,
           W3, b3, g3, be3):
    N = x.shape[0]
    E = edge_index.shape[1]
    G = 64
    src = edge_index[0].reshape(NW, -1, IGRP, C)
    dst = edge_index[1].reshape(NW, -1, IGRP, C)
    batch_r = batch.reshape(N // BLK, 1, BLK)

    deg_parts = _deg_fn(N, E)(dst)
    t, dinv8 = _prep_call(x, W1, deg_parts)

    gs = _gather_scatter_fn(N, E)
    s = gs(t, src, dst)
    t = _bn_layer_call(s, t, dinv8, b1, g1, be1, W2)
    s = gs(t, src, dst)
    t = _bn_layer_call(s, t, dinv8, b2, g2, be2, W3)
    s = gs(t, src, dst)
    return _bn_layer_call(s, t, dinv8, b3, g3, be3, None, batch_r, G)
